# Initial kernel scaffold; baseline (speedup 1.0000x reference)
import jax
import jax.numpy as jnp
from jax.experimental import pallas as pl


def kernel(x, W):
    raise NotImplementedError("write your pallas kernel here")


# speedup vs baseline: 1.1042x; 1.1042x over previous
import functools

import jax
import jax.numpy as jnp
from jax import lax
from jax.experimental import pallas as pl
from jax.experimental.pallas import tpu as pltpu
from jax.experimental.pallas import tpu_sc as plsc

VOCAB = 1000000
EMBED = 32
BATCH = 16384
HIST = 50

NUM_CORES = 2
NUM_SUBCORES = 16
NW = NUM_CORES * NUM_SUBCORES
B = BATCH * HIST
BPW = B // NW
CHUNK = 1600
NCHUNK = BPW // CHUNK

_mesh = plsc.VectorSubcoreMesh(core_axis_name="c", subcore_axis_name="s")


@functools.partial(
    pl.kernel,
    mesh=_mesh,
    compiler_params=pltpu.CompilerParams(use_tc_tiling_on_sc=False),
    out_type=jax.ShapeDtypeStruct((B, EMBED), jnp.float32),
    scratch_types=[
        pltpu.VMEM((CHUNK,), jnp.int32),
        pltpu.VMEM((CHUNK, EMBED), jnp.float32),
        pltpu.SemaphoreType.DMA,
    ],
)
def _gather(idx_hbm, table_hbm, out_hbm, idx_v, rows_v, sem):
    wid = lax.axis_index("s") * NUM_CORES + lax.axis_index("c")
    base = wid * BPW

    def chunk_body(i, carry):
        off = base + i * CHUNK
        pltpu.sync_copy(idx_hbm.at[pl.ds(off, CHUNK)], idx_v)
        pltpu.async_copy(table_hbm.at[idx_v], rows_v, sem).wait()
        pltpu.sync_copy(rows_v, out_hbm.at[pl.ds(off, CHUNK)])
        return carry

    lax.fori_loop(0, NCHUNK, chunk_body, 0)


def kernel(x, W):
    idx = x.reshape(B).astype(jnp.int32)
    out = _gather(idx, W)
    return out.reshape(BATCH, HIST, EMBED)


# speedup vs baseline: 1.1118x; 1.0068x over previous
import functools

import jax
import jax.numpy as jnp
from jax import lax
from jax.experimental import pallas as pl
from jax.experimental.pallas import tpu as pltpu
from jax.experimental.pallas import tpu_sc as plsc

VOCAB = 1000000
EMBED = 32
BATCH = 16384
HIST = 50

NUM_CORES = 2
NUM_SUBCORES = 16
NW = NUM_CORES * NUM_SUBCORES
B = BATCH * HIST
BPW = B // NW
CHUNK = 800
NCHUNK = BPW // CHUNK
NSLOT = 4

_mesh = plsc.VectorSubcoreMesh(core_axis_name="c", subcore_axis_name="s")


@functools.partial(
    pl.kernel,
    mesh=_mesh,
    compiler_params=pltpu.CompilerParams(use_tc_tiling_on_sc=False),
    out_type=jax.ShapeDtypeStruct((B, EMBED), jnp.float32),
    scratch_types=[
        pltpu.VMEM((BPW,), jnp.int32),
        pltpu.VMEM((NSLOT, CHUNK, EMBED), jnp.float32),
        [pltpu.SemaphoreType.DMA] * NSLOT,
        [pltpu.SemaphoreType.DMA] * NSLOT,
    ],
)
def _gather(idx_hbm, table_hbm, out_hbm, idx_v, rows_v, gsems, wsems):
    wid = lax.axis_index("s") * NUM_CORES + lax.axis_index("c")
    base = wid * BPW

    pltpu.sync_copy(idx_hbm.at[pl.ds(base, BPW)], idx_v)

    def gather_start(i):
        return pltpu.async_copy(
            table_hbm.at[idx_v.at[pl.ds(i * CHUNK, CHUNK)]],
            rows_v.at[i % NSLOT], gsems[i % NSLOT])

    def wb_copy(i):
        return pltpu.make_async_copy(
            rows_v.at[i % NSLOT],
            out_hbm.at[pl.ds(base + i * CHUNK, CHUNK)], wsems[i % NSLOT])

    g0 = gather_start(0)
    g1 = gather_start(1)
    pending = {0: g0, 1: g1}
    for i in range(NCHUNK):
        pending.pop(i).wait()
        wb_copy(i).start()
        if i >= 2:
            wb_copy(i - 2).wait()
        if i + 2 < NCHUNK:
            pending[i + 2] = gather_start(i + 2)
    wb_copy(NCHUNK - 2).wait()
    wb_copy(NCHUNK - 1).wait()


def kernel(x, W):
    idx = x.reshape(B).astype(jnp.int32)
    out = _gather(idx, W)
    return out.reshape(BATCH, HIST, EMBED)


# speedup vs baseline: 1.8129x; 1.6306x over previous
import functools

import jax
import jax.numpy as jnp
from jax import lax
from jax.experimental import pallas as pl
from jax.experimental.pallas import tpu as pltpu
from jax.experimental.pallas import tpu_sc as plsc

VOCAB = 1000000
EMBED = 32
BATCH = 16384
HIST = 50

NUM_CORES = 2
NUM_SUBCORES = 16
NW = NUM_CORES * NUM_SUBCORES
BPW = BATCH // NW

_mesh = plsc.VectorSubcoreMesh(core_axis_name="c", subcore_axis_name="s")


@functools.partial(
    pl.kernel,
    mesh=_mesh,
    compiler_params=pltpu.CompilerParams(use_tc_tiling_on_sc=False),
    out_type=jax.ShapeDtypeStruct((BATCH, HIST, EMBED), jnp.float32),
    scratch_types=[
        pltpu.VMEM((HIST, BPW), jnp.int32),
        pltpu.VMEM((4, BPW, EMBED), jnp.float32),
        [pltpu.SemaphoreType.DMA] * 4,
        [pltpu.SemaphoreType.DMA] * 4,
    ],
)
def _gather(idx_hbm, table_hbm, out_hbm, idx_v, rows_v, gsems, wsems):
    wid = lax.axis_index("s") * NUM_CORES + lax.axis_index("c")
    b0 = wid * BPW

    pltpu.sync_copy(idx_hbm.at[:, pl.ds(b0, BPW)], idx_v)

    def gather_start(h):
        return pltpu.async_copy(
            table_hbm.at[idx_v.at[h]], rows_v.at[h % 4], gsems[h % 4])

    def wb_copy(h):
        return pltpu.make_async_copy(
            rows_v.at[h % 4], out_hbm.at[pl.ds(b0, BPW), h], wsems[h % 4])

    pending = {0: gather_start(0), 1: gather_start(1)}
    for h in range(HIST):
        pending.pop(h).wait()
        wb_copy(h).start()
        if h >= 2:
            wb_copy(h - 2).wait()
        if h + 2 < HIST:
            pending[h + 2] = gather_start(h + 2)
    wb_copy(HIST - 2).wait()
    wb_copy(HIST - 1).wait()


def kernel(x, W):
    idx_t = jnp.swapaxes(x, 0, 1).astype(jnp.int32)
    return _gather(idx_t, W)


# speedup vs baseline: 1.8412x; 1.0156x over previous
import functools

import jax
import jax.numpy as jnp
from jax import lax
from jax.experimental import pallas as pl
from jax.experimental.pallas import tpu as pltpu
from jax.experimental.pallas import tpu_sc as plsc

VOCAB = 1000000
EMBED = 32
BATCH = 16384
HIST = 50

NUM_CORES = 2
NUM_SUBCORES = 16
NW = NUM_CORES * NUM_SUBCORES
BPW = BATCH // NW

_mesh = plsc.VectorSubcoreMesh(core_axis_name="c", subcore_axis_name="s")


@functools.partial(
    pl.kernel,
    mesh=_mesh,
    compiler_params=pltpu.CompilerParams(use_tc_tiling_on_sc=False),
    out_type=jax.ShapeDtypeStruct((BATCH, HIST, EMBED), jnp.float32),
    scratch_types=[
        pltpu.VMEM((HIST, BPW), jnp.int32),
        pltpu.VMEM((4, BPW, EMBED), jnp.float32),
        [pltpu.SemaphoreType.DMA] * 4,
        [pltpu.SemaphoreType.DMA] * 4,
    ],
)
def _gather(idx_hbm, table_hbm, out_hbm, idx_v, rows_v, gsems, wsems):
    wid = lax.axis_index("s") * NUM_CORES + lax.axis_index("c")
    b0 = wid * BPW

    pltpu.sync_copy(idx_hbm.at[:, pl.ds(b0, BPW)], idx_v)

    def gather_start(h):
        return pltpu.async_copy(
            table_hbm.at[idx_v.at[h]], rows_v.at[h % 4], gsems[h % 4])

    def wb_copy(h):
        return pltpu.make_async_copy(
            rows_v.at[h % 4], out_hbm.at[pl.ds(b0, BPW), h], wsems[h % 4])

    pending = {0: gather_start(0), 1: gather_start(1)}
    for h in range(HIST):
        pending.pop(h).wait()
        wb_copy(h).start()
        if h >= 2:
            wb_copy(h - 2).wait()
        if h + 2 < HIST:
            pending[h + 2] = gather_start(h + 2)
    wb_copy(HIST - 2).wait()
    wb_copy(HIST - 1).wait()


def kernel(x, W):
    idx_t = jnp.swapaxes(x, 0, 1).astype(jnp.int32) * 4
    W4 = jnp.pad(W, ((0, 0), (0, 128 - EMBED))).reshape(4 * VOCAB, EMBED)
    return _gather(idx_t, W4)


# speedup vs baseline: 2.5983x; 1.4112x over previous
import functools

import jax
import jax.numpy as jnp
from jax import lax
from jax.experimental import pallas as pl
from jax.experimental.pallas import tpu as pltpu
from jax.experimental.pallas import tpu_sc as plsc

VOCAB = 1000000
EMBED = 32
BATCH = 16384
HIST = 50

NUM_CORES = 2
NUM_SUBCORES = 16
NW = NUM_CORES * NUM_SUBCORES
BPW = BATCH // NW

_mesh = plsc.VectorSubcoreMesh(core_axis_name="c", subcore_axis_name="s")


@functools.partial(
    pl.kernel,
    mesh=_mesh,
    compiler_params=pltpu.CompilerParams(use_tc_tiling_on_sc=False),
    out_type=jax.ShapeDtypeStruct((BATCH, 56, 128), jnp.float32),
    scratch_types=[
        pltpu.VMEM((HIST, BPW), jnp.int32),
        pltpu.VMEM((4, BPW, EMBED), jnp.float32),
        [pltpu.SemaphoreType.DMA] * 4,
        [pltpu.SemaphoreType.DMA] * 4,
    ],
)
def _gather(idx_hbm, table_hbm, out_hbm, idx_v, rows_v, gsems, wsems):
    wid = lax.axis_index("s") * NUM_CORES + lax.axis_index("c")
    b0 = wid * BPW

    pltpu.sync_copy(idx_hbm.at[:, pl.ds(b0, BPW)], idx_v)

    def gather_start(h):
        return pltpu.async_copy(
            table_hbm.at[idx_v.at[h]], rows_v.at[h % 4], gsems[h % 4])

    def wb_copy(h):
        return pltpu.make_async_copy(
            rows_v.at[h % 4],
            out_hbm.at[pl.ds(b0, BPW), h, pl.ds(0, EMBED)], wsems[h % 4])

    pending = {0: gather_start(0), 1: gather_start(1)}
    for h in range(HIST):
        pending.pop(h).wait()
        wb_copy(h).start()
        if h >= 2:
            wb_copy(h - 2).wait()
        if h + 2 < HIST:
            pending[h + 2] = gather_start(h + 2)
    wb_copy(HIST - 2).wait()
    wb_copy(HIST - 1).wait()


def kernel(x, W):
    idx_t = jnp.swapaxes(x, 0, 1).astype(jnp.int32) * 4
    W4 = jnp.pad(W, ((0, 0), (0, 128 - EMBED))).reshape(4 * VOCAB, EMBED)
    out_big = _gather(idx_t, W4)
    return out_big[:, :HIST, :EMBED]
